# Initial kernel scaffold; baseline (speedup 1.0000x reference)
import jax
import jax.numpy as jnp
from jax.experimental import pallas as pl


def kernel(x, Wr, br, W1, b1, W2, b2, sW1, sb1, sW2, sb2, alpha, beta, noise):
    raise NotImplementedError("write your pallas kernel here")


# speedup vs baseline: 2.0055x; 2.0055x over previous
import functools

import jax
import jax.numpy as jnp
from jax.experimental import pallas as pl
from jax.experimental.pallas import tpu as pltpu

T = 2048
D = 1024
H = 1024
E = 8
NEG_SLOPE = 0.01


def _moe_body(x_ref, Wr_ref, br_ref, noise_ref, W1_ref, b1_ref, W2_ref,
              b2_ref, sW1_ref, sb1_ref, sW2_ref, sb2_ref, w_ref,
              out_ref, xb_ref, gates_ref):
    j = pl.program_id(0)

    @pl.when(j == 0)
    def _router_and_shared():
        xf = x_ref[...]
        xb_ref[...] = xf.astype(jnp.bfloat16)
        logits = (jnp.dot(xf, Wr_ref[...], preferred_element_type=jnp.float32)
                  + br_ref[...] + noise_ref[...])
        lane = jax.lax.broadcasted_iota(jnp.int32, (T, E), 1)
        v1 = jnp.max(logits, axis=-1, keepdims=True)
        i1 = jnp.min(jnp.where(logits == v1, lane, E), axis=-1, keepdims=True)
        m1 = lane == i1
        l2 = jnp.where(m1, -jnp.inf, logits)
        v2 = jnp.max(l2, axis=-1, keepdims=True)
        i2 = jnp.min(jnp.where(l2 == v2, lane, E), axis=-1, keepdims=True)
        m2 = lane == i2
        e2 = jnp.exp(v2 - v1)
        denom = 1.0 + e2
        g1 = 1.0 / denom
        g2 = e2 / denom
        gates_ref[...] = w_ref[1] * (jnp.where(m1, g1, 0.0)
                                     + jnp.where(m2, g2, 0.0))
        h = (jnp.dot(xb_ref[...], sW1_ref[...].astype(jnp.bfloat16),
                     preferred_element_type=jnp.float32) + sb1_ref[...])
        h = jnp.where(h > 0, h, NEG_SLOPE * h).astype(jnp.bfloat16)
        y = (jnp.dot(h, sW2_ref[...].astype(jnp.bfloat16),
                     preferred_element_type=jnp.float32) + sb2_ref[...])
        out_ref[...] = w_ref[0] * y

    @pl.when(j > 0)
    def _expert():
        lane = jax.lax.broadcasted_iota(jnp.int32, (T, E), 1)
        sel = jnp.sum(jnp.where(lane == (j - 1), gates_ref[...], 0.0),
                      axis=-1, keepdims=True)
        h = (jnp.dot(xb_ref[...], W1_ref[0].astype(jnp.bfloat16),
                     preferred_element_type=jnp.float32) + b1_ref[0])
        h = jnp.where(h > 0, h, NEG_SLOPE * h).astype(jnp.bfloat16)
        y = (jnp.dot(h, W2_ref[0].astype(jnp.bfloat16),
                     preferred_element_type=jnp.float32) + b2_ref[0])
        out_ref[...] += sel * y


@jax.jit
def kernel(x, Wr, br, W1, b1, W2, b2, sW1, sb1, sW2, sb2, alpha, beta, noise):
    w = jax.nn.softmax(jnp.stack([alpha, beta]))
    ew = lambda idx_map, blk: pl.BlockSpec(blk, idx_map)
    full = lambda shape: pl.BlockSpec(shape, lambda j: (0,) * len(shape))
    eidx = lambda j: (jnp.maximum(j - 1, 0), 0, 0)
    out = pl.pallas_call(
        _moe_body,
        grid=(E + 1,),
        in_specs=[
            full((T, D)),
            full((D, E)),
            full((1, E)),
            full((T, E)),
            ew(eidx, (1, D, H)),
            ew(eidx, (1, 1, H)),
            ew(eidx, (1, H, D)),
            ew(eidx, (1, 1, D)),
            full((D, H)),
            full((1, H)),
            full((H, D)),
            full((1, D)),
            pl.BlockSpec(memory_space=pltpu.SMEM),
        ],
        out_specs=full((T, D)),
        out_shape=jax.ShapeDtypeStruct((T, D), jnp.float32),
        scratch_shapes=[
            pltpu.VMEM((T, D), jnp.bfloat16),
            pltpu.VMEM((T, E), jnp.float32),
        ],
        compiler_params=pltpu.CompilerParams(
            dimension_semantics=("arbitrary",)),
    )(x, Wr, br.reshape(1, E), noise, W1, b1.reshape(E, 1, H), W2,
      b2.reshape(E, 1, D), sW1, sb1.reshape(1, H), sW2, sb2.reshape(1, D), w)
    return out
